# baseline (speedup 1.0000x reference)
import functools

import jax
import jax.numpy as jnp
from jax import lax
from jax.experimental import pallas as pl
from jax.experimental.pallas import tpu as pltpu
from jax.experimental.pallas import tpu_sc as plsc

_SC_GRAPHS = 16
_NW = 32


def _scores_kernel(x_ref, w_ref, b_ref, out_ref):
    out_ref[...] = jnp.dot(x_ref[...], w_ref[...],
                           preferred_element_type=jnp.float32) + b_ref[0, 0]


def _tc_kernel(x_ref, path_ref, w2_ref, b2_ref, out_ref):
    li = path_ref.shape[1]
    cat = jnp.dot(x_ref[...], w2_ref[...],
                  preferred_element_type=jnp.float32) + b2_ref[...]
    path2d = path_ref[0].reshape(li * li, li)
    red = lax.dot_general(
        cat, path2d, (((0,), (1,)), ((), ())),
        preferred_element_type=jnp.float32)
    out = red[0:1, :] / (red[1:2, :] + 1e-8)
    out_ref[0] = out.reshape(li, li)


def _sc_row(buf_ref, scv_ref, out_ref, out_off, li):
    nj = li // 16
    jbase = [
        jnp.asarray((j0 * 16 + jnp.arange(16)) * li, dtype=jnp.int32)
        for j0 in range(nj)
    ]
    zero = jnp.zeros((16,), jnp.float32)

    def kbody(k, carry):
        ns, ds = carry
        scv = plsc.load_gather(scv_ref, [jnp.full((16,), k, jnp.int32)])
        new_ns, new_ds = [], []
        for j0 in range(nj):
            v = plsc.load_gather(buf_ref, [jbase[j0] + k])
            new_ns.append(ns[j0] + v * scv)
            new_ds.append(ds[j0] + v)
        return tuple(new_ns), tuple(new_ds)

    ns, ds = lax.fori_loop(0, li, kbody,
                           (tuple([zero] * nj), tuple([zero] * nj)))
    for j0 in range(nj):
        out_ref[pl.ds(out_off + j0 * 16, 16)] = ns[j0] / (ds[j0] + 1e-8)


def _sc_kernel(rows_per_w, li, g0,
               paths_hbm, scores_hbm, out_hbm,
               buf0_v, buf1_v, scv_v, out_v, sem0, sem1):
    wid = lax.axis_index("s") * 2 + lax.axis_index("c")
    r0 = wid * rows_per_w
    gr0 = g0 * li + r0
    g = g0 + r0 // li

    pltpu.sync_copy(scores_hbm.at[g], scv_v)
    pltpu.async_copy(paths_hbm.at[gr0], buf0_v, sem0)
    pltpu.async_copy(paths_hbm.at[gr0 + 1], buf1_v, sem1)

    def body(it, _):
        r = 2 * it

        pltpu.make_async_copy(paths_hbm.at[gr0], buf0_v, sem0).wait()
        _sc_row(buf0_v, scv_v, out_v, r * li, li)

        @pl.when(r + 2 < rows_per_w)
        def _():
            pltpu.async_copy(paths_hbm.at[gr0 + r + 2], buf0_v, sem0)

        pltpu.make_async_copy(paths_hbm.at[gr0], buf1_v, sem1).wait()
        _sc_row(buf1_v, scv_v, out_v, (r + 1) * li, li)

        @pl.when(r + 3 < rows_per_w)
        def _():
            pltpu.async_copy(paths_hbm.at[gr0 + r + 3], buf1_v, sem1)

        return 0

    lax.fori_loop(0, rows_per_w // 2, body, 0)
    pltpu.sync_copy(out_v, out_hbm.at[pl.ds(r0 * li, rows_per_w * li)])


def kernel(x, node_paths, ptr, W, b):
    del ptr
    Bg, Li = node_paths.shape[0], node_paths.shape[1]
    D = x.shape[1]
    g_sc = _SC_GRAPHS
    g_tc = Bg - g_sc

    scores = pl.pallas_call(
        _scores_kernel,
        grid=(1,),
        in_specs=[
            pl.BlockSpec((Bg * Li, D), lambda i: (0, 0)),
            pl.BlockSpec((D, 1), lambda i: (0, 0)),
            pl.BlockSpec((1, 1), lambda i: (0, 0)),
        ],
        out_specs=pl.BlockSpec((Bg * Li, 1), lambda i: (0, 0)),
        out_shape=jax.ShapeDtypeStruct((Bg * Li, 1), jnp.float32),
    )(x, W.reshape(D, 1), b.reshape(1, 1)).reshape(Bg, Li)

    rows_per_w = g_sc * Li // _NW
    paths2d = node_paths.reshape(Bg * Li, Li * Li)
    mesh = plsc.VectorSubcoreMesh(core_axis_name="c", subcore_axis_name="s")
    sc_out = pl.kernel(
        functools.partial(_sc_kernel, rows_per_w, Li, g_tc),
        mesh=mesh,
        out_type=jax.ShapeDtypeStruct((g_sc * Li * Li,), jnp.float32),
        scratch_types=[
            pltpu.VMEM((Li * Li,), jnp.float32),
            pltpu.VMEM((Li * Li,), jnp.float32),
            pltpu.VMEM((Li,), jnp.float32),
            pltpu.VMEM((rows_per_w * Li,), jnp.float32),
            pltpu.SemaphoreType.DMA,
            pltpu.SemaphoreType.DMA,
        ],
        compiler_params=pltpu.CompilerParams(needs_layout_passes=False),
    )(paths2d, scores)
    sc_out = sc_out.reshape(g_sc, Li, Li)

    if g_tc == 0:
        return sc_out

    W2 = jnp.concatenate([W.T, jnp.zeros((D, 1), jnp.float32)], axis=1)
    b2 = jnp.stack([b[0], jnp.float32(1.0)]).reshape(1, 2)
    tc_out = pl.pallas_call(
        _tc_kernel,
        grid=(g_tc,),
        in_specs=[
            pl.BlockSpec((Li, D), lambda g: (g, 0)),
            pl.BlockSpec((1, Li, Li, Li), lambda g: (g, 0, 0, 0)),
            pl.BlockSpec((D, 2), lambda g: (0, 0)),
            pl.BlockSpec((1, 2), lambda g: (0, 0)),
        ],
        out_specs=pl.BlockSpec((1, Li, Li), lambda g: (g, 0, 0)),
        out_shape=jax.ShapeDtypeStruct((g_tc, Li, Li), jnp.float32),
        compiler_params=pltpu.CompilerParams(
            dimension_semantics=("parallel",)),
    )(x, node_paths, W2, b2)

    return jnp.concatenate([tc_out, sc_out], axis=0)


# speedup vs baseline: 1.1074x; 1.1074x over previous
import functools

import jax
import jax.numpy as jnp
from jax import lax
from jax.experimental import pallas as pl
from jax.experimental.pallas import tpu as pltpu
from jax.experimental.pallas import tpu_sc as plsc

_SC_GRAPHS = 16
_NW = 32


def _scores_kernel(x_ref, w_ref, b_ref, out_ref):
    out_ref[...] = jnp.dot(x_ref[...], w_ref[...],
                           preferred_element_type=jnp.float32) + b_ref[0, 0]


def _tc_kernel(x_ref, path_ref, w2_ref, b2_ref, out_ref):
    li = path_ref.shape[1]
    cat = jnp.dot(x_ref[...], w2_ref[...],
                  preferred_element_type=jnp.float32) + b2_ref[...]
    path2d = path_ref[0].reshape(li * li, li)
    red = lax.dot_general(
        cat, path2d, (((0,), (1,)), ((), ())),
        preferred_element_type=jnp.float32)
    out = red[0:1, :] / (red[1:2, :] + 1e-8)
    out_ref[0] = out.reshape(li, li)


def _sc_row(buf_ref, scv_ref, out_ref, out_off, li):
    nj = li // 16
    jbase = [
        jnp.asarray((j0 * 16 + jnp.arange(16)) * li, dtype=jnp.int32)
        for j0 in range(nj)
    ]
    zero = jnp.zeros((16,), jnp.float32)

    unroll = 16

    def kbody(c, carry):
        ns, ds = carry
        ns, ds = list(ns), list(ds)
        scw = scv_ref[pl.ds(c * unroll, 16)]
        for kk in range(unroll):
            k = c * unroll + kk
            sck = scw[kk]
            for j0 in range(nj):
                v = plsc.load_gather(buf_ref, [jbase[j0] + k])
                ns[j0] = ns[j0] + v * sck
                ds[j0] = ds[j0] + v
        return tuple(ns), tuple(ds)

    ns, ds = lax.fori_loop(0, li // unroll, kbody,
                           (tuple([zero] * nj), tuple([zero] * nj)))
    for j0 in range(nj):
        out_ref[pl.ds(out_off + j0 * 16, 16)] = ns[j0] / (ds[j0] + 1e-8)


def _sc_kernel(rows_per_w, li, g0,
               paths_hbm, scores_hbm, out_hbm,
               buf0_v, buf1_v, scv_v, out_v, sem0, sem1):
    wid = lax.axis_index("s") * 2 + lax.axis_index("c")
    r0 = wid * rows_per_w
    gr0 = g0 * li + r0
    g = g0 + r0 // li

    pltpu.sync_copy(scores_hbm.at[g], scv_v)
    pltpu.async_copy(paths_hbm.at[gr0], buf0_v, sem0)
    pltpu.async_copy(paths_hbm.at[gr0 + 1], buf1_v, sem1)

    def body(it, _):
        r = 2 * it

        pltpu.make_async_copy(paths_hbm.at[gr0], buf0_v, sem0).wait()
        _sc_row(buf0_v, scv_v, out_v, r * li, li)

        @pl.when(r + 2 < rows_per_w)
        def _():
            pltpu.async_copy(paths_hbm.at[gr0 + r + 2], buf0_v, sem0)

        pltpu.make_async_copy(paths_hbm.at[gr0], buf1_v, sem1).wait()
        _sc_row(buf1_v, scv_v, out_v, (r + 1) * li, li)

        @pl.when(r + 3 < rows_per_w)
        def _():
            pltpu.async_copy(paths_hbm.at[gr0 + r + 3], buf1_v, sem1)

        return 0

    lax.fori_loop(0, rows_per_w // 2, body, 0)
    pltpu.sync_copy(out_v, out_hbm.at[pl.ds(r0 * li, rows_per_w * li)])


def kernel(x, node_paths, ptr, W, b):
    del ptr
    Bg, Li = node_paths.shape[0], node_paths.shape[1]
    D = x.shape[1]
    g_sc = _SC_GRAPHS
    g_tc = Bg - g_sc

    scores = pl.pallas_call(
        _scores_kernel,
        grid=(1,),
        in_specs=[
            pl.BlockSpec((Bg * Li, D), lambda i: (0, 0)),
            pl.BlockSpec((D, 1), lambda i: (0, 0)),
            pl.BlockSpec((1, 1), lambda i: (0, 0)),
        ],
        out_specs=pl.BlockSpec((Bg * Li, 1), lambda i: (0, 0)),
        out_shape=jax.ShapeDtypeStruct((Bg * Li, 1), jnp.float32),
    )(x, W.reshape(D, 1), b.reshape(1, 1)).reshape(Bg, Li)

    rows_per_w = g_sc * Li // _NW
    paths2d = node_paths.reshape(Bg * Li, Li * Li)
    mesh = plsc.VectorSubcoreMesh(core_axis_name="c", subcore_axis_name="s")
    sc_out = pl.kernel(
        functools.partial(_sc_kernel, rows_per_w, Li, g_tc),
        mesh=mesh,
        out_type=jax.ShapeDtypeStruct((g_sc * Li * Li,), jnp.float32),
        scratch_types=[
            pltpu.VMEM((Li * Li,), jnp.float32),
            pltpu.VMEM((Li * Li,), jnp.float32),
            pltpu.VMEM((Li,), jnp.float32),
            pltpu.VMEM((rows_per_w * Li,), jnp.float32),
            pltpu.SemaphoreType.DMA,
            pltpu.SemaphoreType.DMA,
        ],
        compiler_params=pltpu.CompilerParams(needs_layout_passes=False),
    )(paths2d, scores)
    sc_out = sc_out.reshape(g_sc, Li, Li)

    if g_tc == 0:
        return sc_out

    W2 = jnp.concatenate([W.T, jnp.zeros((D, 1), jnp.float32)], axis=1)
    b2 = jnp.stack([b[0], jnp.float32(1.0)]).reshape(1, 2)
    tc_out = pl.pallas_call(
        _tc_kernel,
        grid=(g_tc,),
        in_specs=[
            pl.BlockSpec((Li, D), lambda g: (g, 0)),
            pl.BlockSpec((1, Li, Li, Li), lambda g: (g, 0, 0, 0)),
            pl.BlockSpec((D, 2), lambda g: (0, 0)),
            pl.BlockSpec((1, 2), lambda g: (0, 0)),
        ],
        out_specs=pl.BlockSpec((1, Li, Li), lambda g: (g, 0, 0)),
        out_shape=jax.ShapeDtypeStruct((g_tc, Li, Li), jnp.float32),
        compiler_params=pltpu.CompilerParams(
            dimension_semantics=("parallel",)),
    )(x, node_paths, W2, b2)

    return jnp.concatenate([tc_out, sc_out], axis=0)


# speedup vs baseline: 3.0537x; 2.7575x over previous
import functools

import jax
import jax.numpy as jnp
from jax import lax
from jax.experimental import pallas as pl
from jax.experimental.pallas import tpu as pltpu
from jax.experimental.pallas import tpu_sc as plsc

_SC_GRAPHS = 16
_NW = 32


def _scores_kernel(x_ref, w_ref, b_ref, out_ref):
    out_ref[...] = jnp.dot(x_ref[...], w_ref[...],
                           preferred_element_type=jnp.float32) + b_ref[0, 0]


def _tc_kernel(x_ref, path_ref, w2_ref, b2_ref, out_ref):
    li = path_ref.shape[1]
    cat = jnp.dot(x_ref[...], w2_ref[...],
                  preferred_element_type=jnp.float32) + b2_ref[...]
    path2d = path_ref[0].reshape(li * li, li)
    red = lax.dot_general(
        cat, path2d, (((0,), (1,)), ((), ())),
        preferred_element_type=jnp.float32)
    out = red[0:1, :] / (red[1:2, :] + 1e-8)
    out_ref[0] = out.reshape(li, li)


def _sc_row(buf_ref, scv_ref, out_ref, out_off, li):
    nj = li // 16
    jbase = [
        jnp.asarray((j0 * 16 + jnp.arange(16)) * li, dtype=jnp.int32)
        for j0 in range(nj)
    ]
    zero = jnp.zeros((16,), jnp.float32)

    unroll = 8
    lane = jnp.arange(16, dtype=jnp.int32)

    def kbody(c, carry):
        ns, ds = carry
        ns, ds = list(ns), list(ds)
        for kk in range(unroll):
            kv = c * unroll + kk + lane
            kwrap = jnp.where(kv >= li, kv - li, kv)
            scw = plsc.load_gather(scv_ref, [kwrap])
            for j0 in range(nj):
                v = plsc.load_gather(buf_ref, [jbase[j0] + kwrap])
                ns[j0] = ns[j0] + v * scw
                ds[j0] = ds[j0] + v
        return tuple(ns), tuple(ds)

    ns, ds = lax.fori_loop(0, li // unroll, kbody,
                           (tuple([zero] * nj), tuple([zero] * nj)))
    for j0 in range(nj):
        out_ref[pl.ds(out_off + j0 * 16, 16)] = ns[j0] / (ds[j0] + 1e-8)


def _sc_kernel(rows_per_w, li, g0,
               paths_hbm, scores_hbm, out_hbm,
               buf0_v, buf1_v, scv_v, out_v, sem0, sem1):
    wid = lax.axis_index("s") * 2 + lax.axis_index("c")
    r0 = wid * rows_per_w
    gr0 = g0 * li + r0
    g = g0 + r0 // li

    pltpu.sync_copy(scores_hbm.at[g], scv_v)
    pltpu.async_copy(paths_hbm.at[gr0], buf0_v, sem0)
    pltpu.async_copy(paths_hbm.at[gr0 + 1], buf1_v, sem1)

    def body(it, _):
        r = 2 * it

        pltpu.make_async_copy(paths_hbm.at[gr0], buf0_v, sem0).wait()
        _sc_row(buf0_v, scv_v, out_v, r * li, li)

        @pl.when(r + 2 < rows_per_w)
        def _():
            pltpu.async_copy(paths_hbm.at[gr0 + r + 2], buf0_v, sem0)

        pltpu.make_async_copy(paths_hbm.at[gr0], buf1_v, sem1).wait()
        _sc_row(buf1_v, scv_v, out_v, (r + 1) * li, li)

        @pl.when(r + 3 < rows_per_w)
        def _():
            pltpu.async_copy(paths_hbm.at[gr0 + r + 3], buf1_v, sem1)

        return 0

    lax.fori_loop(0, rows_per_w // 2, body, 0)
    pltpu.sync_copy(out_v, out_hbm.at[pl.ds(r0 * li, rows_per_w * li)])


def kernel(x, node_paths, ptr, W, b):
    del ptr
    Bg, Li = node_paths.shape[0], node_paths.shape[1]
    D = x.shape[1]
    g_sc = _SC_GRAPHS
    g_tc = Bg - g_sc

    scores = pl.pallas_call(
        _scores_kernel,
        grid=(1,),
        in_specs=[
            pl.BlockSpec((Bg * Li, D), lambda i: (0, 0)),
            pl.BlockSpec((D, 1), lambda i: (0, 0)),
            pl.BlockSpec((1, 1), lambda i: (0, 0)),
        ],
        out_specs=pl.BlockSpec((Bg * Li, 1), lambda i: (0, 0)),
        out_shape=jax.ShapeDtypeStruct((Bg * Li, 1), jnp.float32),
    )(x, W.reshape(D, 1), b.reshape(1, 1)).reshape(Bg, Li)

    rows_per_w = g_sc * Li // _NW
    paths2d = node_paths.reshape(Bg * Li, Li * Li)
    mesh = plsc.VectorSubcoreMesh(core_axis_name="c", subcore_axis_name="s")
    sc_out = pl.kernel(
        functools.partial(_sc_kernel, rows_per_w, Li, g_tc),
        mesh=mesh,
        out_type=jax.ShapeDtypeStruct((g_sc * Li * Li,), jnp.float32),
        scratch_types=[
            pltpu.VMEM((Li * Li,), jnp.float32),
            pltpu.VMEM((Li * Li,), jnp.float32),
            pltpu.VMEM((Li,), jnp.float32),
            pltpu.VMEM((rows_per_w * Li,), jnp.float32),
            pltpu.SemaphoreType.DMA,
            pltpu.SemaphoreType.DMA,
        ],
        compiler_params=pltpu.CompilerParams(needs_layout_passes=False),
    )(paths2d, scores)
    sc_out = sc_out.reshape(g_sc, Li, Li)

    if g_tc == 0:
        return sc_out

    W2 = jnp.concatenate([W.T, jnp.zeros((D, 1), jnp.float32)], axis=1)
    b2 = jnp.stack([b[0], jnp.float32(1.0)]).reshape(1, 2)
    tc_out = pl.pallas_call(
        _tc_kernel,
        grid=(g_tc,),
        in_specs=[
            pl.BlockSpec((Li, D), lambda g: (g, 0)),
            pl.BlockSpec((1, Li, Li, Li), lambda g: (g, 0, 0, 0)),
            pl.BlockSpec((D, 2), lambda g: (0, 0)),
            pl.BlockSpec((1, 2), lambda g: (0, 0)),
        ],
        out_specs=pl.BlockSpec((1, Li, Li), lambda g: (g, 0, 0)),
        out_shape=jax.ShapeDtypeStruct((g_tc, Li, Li), jnp.float32),
        compiler_params=pltpu.CompilerParams(
            dimension_semantics=("parallel",)),
    )(x, node_paths, W2, b2)

    return jnp.concatenate([tc_out, sc_out], axis=0)


# speedup vs baseline: 4.3669x; 1.4300x over previous
import functools

import jax
import jax.numpy as jnp
from jax import lax
from jax.experimental import pallas as pl
from jax.experimental.pallas import tpu as pltpu
from jax.experimental.pallas import tpu_sc as plsc

_SC_GRAPHS = 2
_NW = 32


def _scores_kernel(x_ref, w_ref, b_ref, out_ref):
    out_ref[...] = jnp.dot(x_ref[...], w_ref[...],
                           preferred_element_type=jnp.float32) + b_ref[0, 0]


def _tc_kernel(x_ref, path_ref, w2_ref, b2_ref, out_ref):
    li = path_ref.shape[1]
    cat = jnp.dot(x_ref[...], w2_ref[...],
                  preferred_element_type=jnp.float32) + b2_ref[...]
    path2d = path_ref[0].reshape(li * li, li)
    red = lax.dot_general(
        cat, path2d, (((0,), (1,)), ((), ())),
        preferred_element_type=jnp.float32)
    out = red[0:1, :] / (red[1:2, :] + 1e-8)
    out_ref[0] = out.reshape(li, li)


def _sc_row(buf_ref, scv_ref, out_ref, out_off, li):
    nj = li // 16
    jbase = [
        jnp.asarray((j0 * 16 + jnp.arange(16)) * li, dtype=jnp.int32)
        for j0 in range(nj)
    ]
    zero = jnp.zeros((16,), jnp.float32)

    unroll = 8
    lane = jnp.arange(16, dtype=jnp.int32)

    def kbody(c, carry):
        ns, ds = carry
        ns, ds = list(ns), list(ds)
        for kk in range(unroll):
            kv = c * unroll + kk + lane
            kwrap = jnp.where(kv >= li, kv - li, kv)
            scw = plsc.load_gather(scv_ref, [kwrap])
            for j0 in range(nj):
                v = plsc.load_gather(buf_ref, [jbase[j0] + kwrap])
                ns[j0] = ns[j0] + v * scw
                ds[j0] = ds[j0] + v
        return tuple(ns), tuple(ds)

    ns, ds = lax.fori_loop(0, li // unroll, kbody,
                           (tuple([zero] * nj), tuple([zero] * nj)))
    for j0 in range(nj):
        out_ref[pl.ds(out_off + j0 * 16, 16)] = ns[j0] / (ds[j0] + 1e-8)


def _sc_kernel(rows_per_w, li, g0,
               paths_hbm, scores_hbm, out_hbm,
               buf0_v, buf1_v, scv_v, out_v, sem0, sem1):
    wid = lax.axis_index("s") * 2 + lax.axis_index("c")
    r0 = wid * rows_per_w
    gr0 = g0 * li + r0
    g = g0 + r0 // li

    pltpu.sync_copy(scores_hbm.at[g], scv_v)
    pltpu.async_copy(paths_hbm.at[gr0], buf0_v, sem0)
    pltpu.async_copy(paths_hbm.at[gr0 + 1], buf1_v, sem1)

    def body(it, _):
        r = 2 * it

        pltpu.make_async_copy(paths_hbm.at[gr0], buf0_v, sem0).wait()
        _sc_row(buf0_v, scv_v, out_v, r * li, li)

        @pl.when(r + 2 < rows_per_w)
        def _():
            pltpu.async_copy(paths_hbm.at[gr0 + r + 2], buf0_v, sem0)

        pltpu.make_async_copy(paths_hbm.at[gr0], buf1_v, sem1).wait()
        _sc_row(buf1_v, scv_v, out_v, (r + 1) * li, li)

        @pl.when(r + 3 < rows_per_w)
        def _():
            pltpu.async_copy(paths_hbm.at[gr0 + r + 3], buf1_v, sem1)

        return 0

    lax.fori_loop(0, rows_per_w // 2, body, 0)
    pltpu.sync_copy(out_v, out_hbm.at[pl.ds(r0 * li, rows_per_w * li)])


def kernel(x, node_paths, ptr, W, b):
    del ptr
    Bg, Li = node_paths.shape[0], node_paths.shape[1]
    D = x.shape[1]
    g_sc = _SC_GRAPHS
    g_tc = Bg - g_sc

    scores = pl.pallas_call(
        _scores_kernel,
        grid=(1,),
        in_specs=[
            pl.BlockSpec((Bg * Li, D), lambda i: (0, 0)),
            pl.BlockSpec((D, 1), lambda i: (0, 0)),
            pl.BlockSpec((1, 1), lambda i: (0, 0)),
        ],
        out_specs=pl.BlockSpec((Bg * Li, 1), lambda i: (0, 0)),
        out_shape=jax.ShapeDtypeStruct((Bg * Li, 1), jnp.float32),
    )(x, W.reshape(D, 1), b.reshape(1, 1)).reshape(Bg, Li)

    rows_per_w = g_sc * Li // _NW
    paths2d = node_paths.reshape(Bg * Li, Li * Li)
    mesh = plsc.VectorSubcoreMesh(core_axis_name="c", subcore_axis_name="s")
    sc_out = pl.kernel(
        functools.partial(_sc_kernel, rows_per_w, Li, g_tc),
        mesh=mesh,
        out_type=jax.ShapeDtypeStruct((g_sc * Li * Li,), jnp.float32),
        scratch_types=[
            pltpu.VMEM((Li * Li,), jnp.float32),
            pltpu.VMEM((Li * Li,), jnp.float32),
            pltpu.VMEM((Li,), jnp.float32),
            pltpu.VMEM((rows_per_w * Li,), jnp.float32),
            pltpu.SemaphoreType.DMA,
            pltpu.SemaphoreType.DMA,
        ],
        compiler_params=pltpu.CompilerParams(needs_layout_passes=False),
    )(paths2d, scores)
    sc_out = sc_out.reshape(g_sc, Li, Li)

    if g_tc == 0:
        return sc_out

    W2 = jnp.concatenate([W.T, jnp.zeros((D, 1), jnp.float32)], axis=1)
    b2 = jnp.stack([b[0], jnp.float32(1.0)]).reshape(1, 2)
    tc_out = pl.pallas_call(
        _tc_kernel,
        grid=(g_tc,),
        in_specs=[
            pl.BlockSpec((Li, D), lambda g: (g, 0)),
            pl.BlockSpec((1, Li, Li, Li), lambda g: (g, 0, 0, 0)),
            pl.BlockSpec((D, 2), lambda g: (0, 0)),
            pl.BlockSpec((1, 2), lambda g: (0, 0)),
        ],
        out_specs=pl.BlockSpec((1, Li, Li), lambda g: (g, 0, 0)),
        out_shape=jax.ShapeDtypeStruct((g_tc, Li, Li), jnp.float32),
        compiler_params=pltpu.CompilerParams(
            dimension_semantics=("parallel",)),
    )(x, node_paths, W2, b2)

    return jnp.concatenate([tc_out, sc_out], axis=0)


# speedup vs baseline: 10.3449x; 2.3690x over previous
import functools

import jax
import jax.numpy as jnp
from jax import lax
from jax.experimental import pallas as pl
from jax.experimental.pallas import tpu as pltpu
from jax.experimental.pallas import tpu_sc as plsc

_SC_GRAPHS = 2
_NW = 32


def _scores_kernel(x_ref, w_ref, b_ref, out_ref):
    bg, li = out_ref.shape
    s = jnp.dot(x_ref[...], w_ref[...],
                preferred_element_type=jnp.float32) + b_ref[0, 0]
    out_ref[...] = s.reshape(bg, li)


def _tc_kernel(x_ref, path_ref, w2_ref, b2_ref, out_ref):
    li = path_ref.shape[1]
    cat = jnp.dot(x_ref[...], w2_ref[...],
                  preferred_element_type=jnp.float32) + b2_ref[...]
    path2d = path_ref[0].reshape(li * li, li)
    red = lax.dot_general(
        cat, path2d, (((0,), (1,)), ((), ())),
        preferred_element_type=jnp.float32)
    out = red[0:1, :] / (red[1:2, :] + 1e-8)
    out_ref[0] = out.reshape(li, li)


def _sc_row(buf_ref, scv_ref, out_ref, r_local, li):
    nj = li // 16
    jvec = [jnp.asarray(j0 * 16 + jnp.arange(16), dtype=jnp.int32)
            for j0 in range(nj)]
    zero = jnp.zeros((16,), jnp.float32)
    unroll = 8
    lane = jnp.arange(16, dtype=jnp.int32)

    def kbody(c, carry):
        ns, ds = carry
        ns, ds = list(ns), list(ds)
        for kk in range(unroll):
            kv = c * unroll + kk + lane
            kwrap = jnp.where(kv >= li, kv - li, kv)
            scw = plsc.load_gather(scv_ref, [kwrap])
            for j0 in range(nj):
                v = plsc.load_gather(buf_ref, [jvec[j0], kwrap])
                ns[j0] = ns[j0] + v * scw
                ds[j0] = ds[j0] + v
        return tuple(ns), tuple(ds)

    ns, ds = lax.fori_loop(0, li // unroll, kbody,
                           (tuple([zero] * nj), tuple([zero] * nj)))
    for j0 in range(nj):
        out_ref[r_local, pl.ds(j0 * 16, 16)] = ns[j0] / (ds[j0] + 1e-8)


def _sc_kernel(rows_per_w, li, g0,
               paths_hbm, scores_hbm, out_hbm,
               buf0_v, buf1_v, scv_v, out_v, sem0, sem1):
    wid = lax.axis_index("s") * 2 + lax.axis_index("c")
    r0 = wid * rows_per_w
    gl = r0 // li
    i0 = r0 % li
    g = g0 + gl

    pltpu.sync_copy(scores_hbm.at[g], scv_v)
    pltpu.async_copy(paths_hbm.at[g, i0], buf0_v, sem0)
    pltpu.async_copy(paths_hbm.at[g, i0 + 1], buf1_v, sem1)

    def body(it, _):
        r = 2 * it

        pltpu.make_async_copy(paths_hbm.at[g, i0], buf0_v, sem0).wait()
        _sc_row(buf0_v, scv_v, out_v, r, li)

        @pl.when(r + 2 < rows_per_w)
        def _():
            pltpu.async_copy(paths_hbm.at[g, i0 + r + 2], buf0_v, sem0)

        pltpu.make_async_copy(paths_hbm.at[g, i0], buf1_v, sem1).wait()
        _sc_row(buf1_v, scv_v, out_v, r + 1, li)

        @pl.when(r + 3 < rows_per_w)
        def _():
            pltpu.async_copy(paths_hbm.at[g, i0 + r + 3], buf1_v, sem1)

        return 0

    lax.fori_loop(0, rows_per_w // 2, body, 0)
    pltpu.sync_copy(out_v, out_hbm.at[gl, pl.ds(i0, rows_per_w)])


def kernel(x, node_paths, ptr, W, b):
    del ptr
    Bg, Li = node_paths.shape[0], node_paths.shape[1]
    D = x.shape[1]
    g_sc = _SC_GRAPHS
    g_tc = Bg - g_sc

    scores = pl.pallas_call(
        _scores_kernel,
        grid=(1,),
        in_specs=[
            pl.BlockSpec((Bg * Li, D), lambda i: (0, 0)),
            pl.BlockSpec((D, 1), lambda i: (0, 0)),
            pl.BlockSpec((1, 1), lambda i: (0, 0)),
        ],
        out_specs=pl.BlockSpec((Bg, Li), lambda i: (0, 0)),
        out_shape=jax.ShapeDtypeStruct((Bg, Li), jnp.float32),
    )(x, W.reshape(D, 1), b.reshape(1, 1))

    rows_per_w = g_sc * Li // _NW
    mesh = plsc.VectorSubcoreMesh(core_axis_name="c", subcore_axis_name="s")
    sc_out = pl.kernel(
        functools.partial(_sc_kernel, rows_per_w, Li, g_tc),
        mesh=mesh,
        out_type=jax.ShapeDtypeStruct((g_sc, Li, Li), jnp.float32),
        scratch_types=[
            pltpu.VMEM((Li, Li), jnp.float32),
            pltpu.VMEM((Li, Li), jnp.float32),
            pltpu.VMEM((Li,), jnp.float32),
            pltpu.VMEM((rows_per_w, Li), jnp.float32),
            pltpu.SemaphoreType.DMA,
            pltpu.SemaphoreType.DMA,
        ],
        compiler_params=pltpu.CompilerParams(needs_layout_passes=False),
    )(node_paths, scores)

    if g_tc == 0:
        return sc_out

    W2 = jnp.concatenate([W.T, jnp.zeros((D, 1), jnp.float32)], axis=1)
    b2 = jnp.stack([b[0], jnp.float32(1.0)]).reshape(1, 2)
    tc_out = pl.pallas_call(
        _tc_kernel,
        grid=(g_tc,),
        in_specs=[
            pl.BlockSpec((Li, D), lambda g: (g, 0)),
            pl.BlockSpec((1, Li, Li, Li), lambda g: (g, 0, 0, 0)),
            pl.BlockSpec((D, 2), lambda g: (0, 0)),
            pl.BlockSpec((1, 2), lambda g: (0, 0)),
        ],
        out_specs=pl.BlockSpec((1, Li, Li), lambda g: (g, 0, 0)),
        out_shape=jax.ShapeDtypeStruct((g_tc, Li, Li), jnp.float32),
        compiler_params=pltpu.CompilerParams(
            dimension_semantics=("parallel",)),
    )(x, node_paths, W2, b2)

    return jnp.concatenate([tc_out, sc_out], axis=0)


# speedup vs baseline: 10.3571x; 1.0012x over previous
import functools

import jax
import jax.numpy as jnp
from jax import lax
from jax.experimental import pallas as pl
from jax.experimental.pallas import tpu as pltpu
from jax.experimental.pallas import tpu_sc as plsc

_SC_GRAPHS = 2
_NW = 32


def _scores_kernel(x_ref, w_ref, b_ref, out_ref):
    bg, li = out_ref.shape
    s = jnp.dot(x_ref[...], w_ref[...],
                preferred_element_type=jnp.float32) + b_ref[0, 0]
    out_ref[...] = s.reshape(bg, li)


def _tc_kernel(x_ref, path_ref, w2_ref, b2_ref, out_ref):
    li = path_ref.shape[1]
    cat = jnp.dot(x_ref[...], w2_ref[...],
                  preferred_element_type=jnp.float32) + b2_ref[...]
    path2d = path_ref[0].reshape(li * li, li)
    red = lax.dot_general(
        cat, path2d, (((0,), (1,)), ((), ())),
        preferred_element_type=jnp.float32)
    out = red[0:1, :] / (red[1:2, :] + 1e-8)
    out_ref[0] = out.reshape(li, li)


def _sc_row(buf_ref, scv_ref, out_ref, r_local, li):
    nj = li // 16
    jvec = [jnp.asarray(j0 * 16 + jnp.arange(16), dtype=jnp.int32)
            for j0 in range(nj)]
    zero = jnp.zeros((16,), jnp.float32)
    unroll = 8
    lane = jnp.arange(16, dtype=jnp.int32)

    def kbody(c, carry):
        ns, ds = carry
        ns, ds = list(ns), list(ds)
        for kk in range(unroll):
            kv = c * unroll + kk + lane
            kwrap = jnp.where(kv >= li, kv - li, kv)
            scw = plsc.load_gather(scv_ref, [kwrap])
            for j0 in range(nj):
                v = plsc.load_gather(buf_ref, [jvec[j0], kwrap])
                ns[j0] = ns[j0] + v * scw
                ds[j0] = ds[j0] + v
        return tuple(ns), tuple(ds)

    ns, ds = lax.fori_loop(0, li // unroll, kbody,
                           (tuple([zero] * nj), tuple([zero] * nj)))
    for j0 in range(nj):
        out_ref[r_local, pl.ds(j0 * 16, 16)] = ns[j0] / (ds[j0] + 1e-8)


def _sc_kernel(rows_per_w, li, g0,
               paths_hbm, scores_hbm, out_hbm,
               buf0_v, buf1_v, scv_v, out_v, sem0, sem1):
    wid = lax.axis_index("s") * 2 + lax.axis_index("c")
    r0 = wid * rows_per_w
    gl = r0 // li
    i0 = r0 % li
    g = g0 + gl

    pltpu.sync_copy(scores_hbm.at[g], scv_v)
    pltpu.async_copy(paths_hbm.at[g, i0], buf0_v, sem0)
    pltpu.async_copy(paths_hbm.at[g, i0 + 1], buf1_v, sem1)

    def body(it, _):
        r = 2 * it

        pltpu.make_async_copy(paths_hbm.at[g, i0], buf0_v, sem0).wait()
        _sc_row(buf0_v, scv_v, out_v, r, li)

        @pl.when(r + 2 < rows_per_w)
        def _():
            pltpu.async_copy(paths_hbm.at[g, i0 + r + 2], buf0_v, sem0)

        pltpu.make_async_copy(paths_hbm.at[g, i0], buf1_v, sem1).wait()
        _sc_row(buf1_v, scv_v, out_v, r + 1, li)

        @pl.when(r + 3 < rows_per_w)
        def _():
            pltpu.async_copy(paths_hbm.at[g, i0 + r + 3], buf1_v, sem1)

        return 0

    lax.fori_loop(0, rows_per_w // 2, body, 0)
    pltpu.sync_copy(out_v, out_hbm.at[gl, pl.ds(i0, rows_per_w)])


def kernel(x, node_paths, ptr, W, b):
    del ptr
    Bg, Li = node_paths.shape[0], node_paths.shape[1]
    D = x.shape[1]
    g_sc = _SC_GRAPHS
    g_tc = Bg - g_sc

    scores = pl.pallas_call(
        _scores_kernel,
        grid=(1,),
        in_specs=[
            pl.BlockSpec((Bg * Li, D), lambda i: (0, 0)),
            pl.BlockSpec((D, 1), lambda i: (0, 0)),
            pl.BlockSpec((1, 1), lambda i: (0, 0)),
        ],
        out_specs=pl.BlockSpec((Bg, Li), lambda i: (0, 0)),
        out_shape=jax.ShapeDtypeStruct((Bg, Li), jnp.float32),
    )(x, W.reshape(D, 1), b.reshape(1, 1))

    rows_per_w = g_sc * Li // _NW
    mesh = plsc.VectorSubcoreMesh(core_axis_name="c", subcore_axis_name="s")
    sc_out = pl.kernel(
        functools.partial(_sc_kernel, rows_per_w, Li, 0),
        mesh=mesh,
        out_type=jax.ShapeDtypeStruct((g_sc, Li, Li), jnp.float32),
        scratch_types=[
            pltpu.VMEM((Li, Li), jnp.float32),
            pltpu.VMEM((Li, Li), jnp.float32),
            pltpu.VMEM((Li,), jnp.float32),
            pltpu.VMEM((rows_per_w, Li), jnp.float32),
            pltpu.SemaphoreType.DMA,
            pltpu.SemaphoreType.DMA,
        ],
        compiler_params=pltpu.CompilerParams(needs_layout_passes=False),
    )(node_paths, scores)

    if g_tc == 0:
        return sc_out

    W2 = jnp.concatenate([W.T, jnp.zeros((D, 1), jnp.float32)], axis=1)
    b2 = jnp.stack([b[0], jnp.float32(1.0)]).reshape(1, 2)
    tc_out = pl.pallas_call(
        _tc_kernel,
        grid=(g_tc,),
        in_specs=[
            pl.BlockSpec((Li, D), lambda g: (g + g_sc, 0)),
            pl.BlockSpec((1, Li, Li, Li), lambda g: (g + g_sc, 0, 0, 0)),
            pl.BlockSpec((D, 2), lambda g: (0, 0)),
            pl.BlockSpec((1, 2), lambda g: (0, 0)),
        ],
        out_specs=pl.BlockSpec((1, Li, Li), lambda g: (g, 0, 0)),
        out_shape=jax.ShapeDtypeStruct((g_tc, Li, Li), jnp.float32),
        compiler_params=pltpu.CompilerParams(
            dimension_semantics=("parallel",)),
    )(x, node_paths, W2, b2)

    return jnp.concatenate([sc_out, tc_out], axis=0)
